# baseline (speedup 1.0000x reference)
import functools

import jax
import jax.numpy as jnp
from jax import lax
from jax.experimental import pallas as pl
from jax.experimental.pallas import tpu as pltpu
from jax.experimental.pallas import tpu_sc as plsc

LANES = 16
BLK = 128


def kernel(x, weight):
    batch, hist = x.shape
    vocab, dim = weight.shape
    assert batch % BLK == 0 and dim % 8 == 0

    info = plsc.get_sparse_core_info()
    nw = info.num_cores * info.num_subcores
    nblocks = hist * (batch // BLK)
    per_tile = nblocks // nw
    assert per_tile * nw == nblocks and per_tile % 2 == 0

    xt = jnp.transpose(x.astype(jnp.int32)).reshape(nblocks, BLK)

    o5_rows = hist * (dim // 8) * (batch // BLK)

    mesh = plsc.VectorSubcoreMesh(core_axis_name="c", subcore_axis_name="s")

    @functools.partial(
        pl.kernel,
        out_type=jax.ShapeDtypeStruct((o5_rows * 8 * BLK,), jnp.float32),
        mesh=mesh,
        compiler_params=pltpu.CompilerParams(
            use_tc_tiling_on_sc=False, needs_layout_passes=False
        ),
        scratch_types=[
            pltpu.VMEM((per_tile, BLK), jnp.int32),
            pltpu.VMEM((4, BLK, dim), jnp.float32),
            pltpu.VMEM((2, dim * BLK), jnp.float32),
            pltpu.SemaphoreType.DMA,
            pltpu.SemaphoreType.DMA,
            pltpu.SemaphoreType.DMA,
            pltpu.SemaphoreType.DMA,
            pltpu.SemaphoreType.DMA,
            pltpu.SemaphoreType.DMA,
        ],
    )
    def gather_kernel(
        xt_hbm, table_hbm, o_hbm, idx_v, buf, tbuf,
        gsem0, gsem1, gsem2, gsem3, ssem0, ssem1,
    ):
        wid = lax.axis_index("s") * info.num_cores + lax.axis_index("c")
        base_blk = wid * per_tile
        pltpu.sync_copy(xt_hbm.at[pl.ds(base_blk, per_tile)], idx_v)

        gsems = (gsem0, gsem1, gsem2, gsem3)
        ssems = (ssem0, ssem1)
        iota = lax.iota(jnp.int32, LANES)

        def fire_gather(i, s):
            pltpu.async_copy(table_hbm.at[idx_v.at[i]], buf.at[s], gsems[s])

        def drain_gather(s):
            pltpu.make_async_copy(
                table_hbm.at[pl.ds(0, BLK)], buf.at[s], gsems[s]
            ).wait()

        riota_scaled = tuple((iota + k * LANES) * BLK for k in range(dim // LANES))

        def transpose(sg, st):
            bufs = buf.at[sg]
            tbufs = tbuf.at[st]

            @plsc.parallel_loop(0, BLK, unroll=8, carry=riota_scaled)
            def _(j, idxs):
                for k in range(dim // LANES):
                    vals = bufs[j, pl.ds(k * LANES, LANES)]
                    plsc.store_scatter(tbufs, [idxs[k]], vals)
                return tuple(v + 1 for v in idxs)

        def fire_store(i, s):
            g = base_blk + i
            h = g // BLK
            b_hi = g % BLK
            r0 = (h * (dim // 8)) * BLK + b_hi
            for c_hi in range(dim // 8):
                pltpu.async_copy(
                    tbuf.at[s, pl.ds(c_hi * 8 * BLK, 8 * BLK)],
                    o_hbm.at[pl.ds((r0 + c_hi * BLK) * 8 * BLK, 8 * BLK)],
                    ssems[s],
                )

        def wait_store(s):
            for c_hi in range(dim // 8):
                pltpu.make_async_copy(
                    o_hbm.at[pl.ds(0, 8 * BLK)],
                    tbuf.at[s, pl.ds(c_hi * 8 * BLK, 8 * BLK)],
                    ssems[s],
                ).wait()

        def step(i, sg, st, first, last):
            drain_gather(sg)
            if not first:
                wait_store(st)
            transpose(sg, st)
            fire_store(i, st)
            if not last:
                fire_gather(i + 4, sg)

        def body(i4, carry, first=False, last=False):
            a = 4 * i4
            for k in range(4):
                step(a + k, k, k % 2, first and k < 2, last)
            return carry

        for k in range(4):
            fire_gather(k, k)
        body(0, 0, first=True)
        lax.fori_loop(1, per_tile // 4 - 1, body, 0)
        body(per_tile // 4 - 1, 0, last=True)
        wait_store(0)
        wait_store(1)

    o5 = gather_kernel(xt, weight)
    o5 = o5.reshape(hist, dim // 8, batch // BLK, 8, BLK)
    out = o5.transpose(2, 4, 0, 1, 3).reshape(batch, hist, dim)
    return out


# speedup vs baseline: 1.7394x; 1.7394x over previous
import functools

import jax
import jax.numpy as jnp
from jax import lax
from jax.experimental import pallas as pl
from jax.experimental.pallas import tpu as pltpu
from jax.experimental.pallas import tpu_sc as plsc

LANES = 16
BLK = 128


def kernel(x, weight):
    batch, hist = x.shape
    vocab, dim = weight.shape
    assert batch % BLK == 0 and dim % 8 == 0

    info = plsc.get_sparse_core_info()
    nw = info.num_cores * info.num_subcores
    nblocks = hist * (batch // BLK)
    per_tile = nblocks // nw
    assert per_tile * nw == nblocks and per_tile % 2 == 0

    xt = jnp.transpose(x.astype(jnp.int32)).reshape(nblocks, BLK)

    o5_rows = hist * (dim // 8) * (batch // BLK)

    mesh = plsc.VectorSubcoreMesh(core_axis_name="c", subcore_axis_name="s")

    @functools.partial(
        pl.kernel,
        out_type=jax.ShapeDtypeStruct((o5_rows, 8, BLK), jnp.float32),
        mesh=mesh,
        compiler_params=pltpu.CompilerParams(
            use_tc_tiling_on_sc=False, needs_layout_passes=False
        ),
        scratch_types=[
            pltpu.VMEM((per_tile, BLK), jnp.int32),
            pltpu.VMEM((4, BLK, dim), jnp.float32),
            pltpu.VMEM((2, dim, BLK + 1), jnp.float32),
            pltpu.SemaphoreType.DMA,
            pltpu.SemaphoreType.DMA,
            pltpu.SemaphoreType.DMA,
            pltpu.SemaphoreType.DMA,
            pltpu.SemaphoreType.DMA,
            pltpu.SemaphoreType.DMA,
        ],
    )
    def gather_kernel(
        xt_hbm, table_hbm, o_hbm, idx_v, buf, tbuf,
        gsem0, gsem1, gsem2, gsem3, ssem0, ssem1,
    ):
        wid = lax.axis_index("s") * info.num_cores + lax.axis_index("c")
        base_blk = wid * per_tile
        pltpu.sync_copy(xt_hbm.at[pl.ds(base_blk, per_tile)], idx_v)

        gsems = (gsem0, gsem1, gsem2, gsem3)
        ssems = (ssem0, ssem1)
        iota = lax.iota(jnp.int32, LANES)

        def fire_gather(i, s):
            pltpu.async_copy(table_hbm.at[idx_v.at[i]], buf.at[s], gsems[s])

        def drain_gather(s):
            pltpu.make_async_copy(
                table_hbm.at[pl.ds(0, BLK)], buf.at[s], gsems[s]
            ).wait()

        riota = tuple(iota + k * LANES for k in range(dim // LANES))

        def transpose(sg, st):
            bufs = buf.at[sg]
            tbufs = tbuf.at[st]

            @plsc.parallel_loop(0, BLK, unroll=8)
            def _(j):
                cols = jnp.full((LANES,), j, jnp.int32)
                for k in range(dim // LANES):
                    vals = bufs[j, pl.ds(k * LANES, LANES)]
                    plsc.store_scatter(tbufs, [riota[k], cols], vals)

        def fire_store(i, s):
            g = base_blk + i
            h = g // BLK
            b_hi = g % BLK
            r0 = (h * (dim // 8)) * BLK + b_hi
            for c_hi in range(dim // 8):
                pltpu.async_copy(
                    tbuf.at[s, pl.ds(c_hi * 8, 8), pl.ds(0, BLK)],
                    o_hbm.at[r0 + c_hi * BLK],
                    ssems[s],
                )

        def wait_store(s):
            for c_hi in range(dim // 8):
                pltpu.make_async_copy(
                    o_hbm.at[0],
                    tbuf.at[s, pl.ds(c_hi * 8, 8), pl.ds(0, BLK)],
                    ssems[s],
                ).wait()

        def step(i, sg, st, first, last):
            drain_gather(sg)
            if not first:
                wait_store(st)
            transpose(sg, st)
            fire_store(i, st)
            if not last:
                fire_gather(i + 4, sg)

        def body(i4, carry, first=False, last=False):
            a = 4 * i4
            for k in range(4):
                step(a + k, k, k % 2, first and k < 2, last)
            return carry

        for k in range(4):
            fire_gather(k, k)
        body(0, 0, first=True)
        lax.fori_loop(1, per_tile // 4 - 1, body, 0)
        body(per_tile // 4 - 1, 0, last=True)
        wait_store(0)
        wait_store(1)

    o5 = gather_kernel(xt, weight)
    o5 = o5.reshape(hist, dim // 8, batch // BLK, 8, BLK)
    out = o5.transpose(2, 4, 0, 1, 3).reshape(batch, hist, dim)
    return out
